# baseline (speedup 1.0000x reference)
import functools

import jax
import jax.numpy as jnp
from jax import lax
from jax.experimental import pallas as pl
from jax.experimental.pallas import tpu as pltpu
from jax.experimental.pallas import tpu_sc as plsc

_B, _CIN, _H, _W = 4, 96, 224, 224
_COUT, _KH, _KW, _SP = 192, 3, 3, 3
_DICT = 256
_HO, _WO = 112, 112
_BH = 16


def _sc_build_m(idx_p, cf_p):
    mesh = plsc.VectorSubcoreMesh(core_axis_name="c", subcore_axis_name="s")

    @functools.partial(
        pl.kernel,
        mesh=mesh,
        compiler_params=pltpu.CompilerParams(needs_layout_passes=False),
        out_type=jax.ShapeDtypeStruct((256 * 256,), jnp.float32),
        scratch_types=[
            pltpu.VMEM((8, 32), jnp.int32),
            pltpu.VMEM((8, 32), jnp.float32),
            pltpu.VMEM((8 * 256,), jnp.float32),
        ],
    )
    def build(idx_hbm, cf_hbm, m_hbm, idx_v, cf_v, buf):
        wid = lax.axis_index("s") * 2 + lax.axis_index("c")
        base = wid * 8
        pltpu.sync_copy(idx_hbm.at[pl.ds(base, 8)], idx_v)
        pltpu.sync_copy(cf_hbm.at[pl.ds(base, 8)], cf_v)
        z = jnp.zeros((16,), jnp.float32)
        for i in range(128):
            buf[pl.ds(i * 16, 16)] = z
        lanes = lax.broadcasted_iota(jnp.int32, (16,), 0)
        masks = [lanes == j for j in range(16)]
        for r in range(8):
            for v in range(2):
                iv = idx_v[r, pl.ds(v * 16, 16)] + r * 256
                cv = cf_v[r, pl.ds(v * 16, 16)]
                for j in range(16):
                    plsc.addupdate_scatter(buf, [iv], cv, mask=masks[j])
        pltpu.sync_copy(buf, m_hbm.at[pl.ds(wid * 2048, 2048)])

    return build(idx_p, cf_p)


def _conv_body(mt_ref, dr_ref, pr0a, pr1a, pr0b, q0a, q1a, q0b, out_ref, w_ref):
    b = pl.program_id(0)
    h = pl.program_id(1)

    @pl.when((b == 0) & (h == 0))
    def _():
        w_ref[...] = jnp.dot(
            dr_ref[...], mt_ref[...], preferred_element_type=jnp.float32
        )

    acc = jnp.zeros((_BH * _WO, _COUT), jnp.float32)
    groups = ((pr0a, q0a), (pr1a, q1a), (pr0b, q0b))
    for kh, (pr, q) in enumerate(groups):
        xp_ = pr[0, :, 0:_WO, :].reshape(_BH * _WO, 2 * _CIN)
        acc = acc + jnp.dot(
            xp_,
            w_ref[kh * 288 : kh * 288 + 192, :],
            preferred_element_type=jnp.float32,
        )
        xq = q[0].reshape(_BH * _WO, _CIN)
        acc = acc + jnp.dot(
            xq,
            w_ref[kh * 288 + 192 : kh * 288 + 288, :],
            preferred_element_type=jnp.float32,
        )
    out_ref[0] = acc.reshape(_BH, _WO, _COUT)


def _conv_tc(mt, dict_r, pr0a, pr1a, pr0b, q0a, q1a, q0b):
    grid = (_B, _HO // _BH)
    pmap = lambda b, h: (b, h, 0, 0)
    const2 = lambda b, h: (0, 0)
    pr_spec = pl.BlockSpec((1, _BH, 113, 2 * _CIN), pmap)
    q_spec = pl.BlockSpec((1, _BH, _WO, _CIN), pmap)
    return pl.pallas_call(
        _conv_body,
        grid=grid,
        in_specs=[
            pl.BlockSpec((_DICT, _COUT), const2),
            pl.BlockSpec((9 * _CIN, _DICT), const2),
            pr_spec, pr_spec, pr_spec,
            q_spec, q_spec, q_spec,
        ],
        out_specs=pl.BlockSpec((1, _BH, _WO, _COUT), pmap),
        out_shape=jax.ShapeDtypeStruct((_B, _HO, _WO, _COUT), jnp.float32),
        scratch_shapes=[pltpu.VMEM((9 * _CIN, _COUT), jnp.float32)],
    )(mt, dict_r, pr0a, pr1a, pr0b, q0a, q1a, q0b)


def kernel(x, dictionary, lookup_indices, lookup_coefficients):
    order = jnp.argsort(-jnp.abs(lookup_coefficients), axis=-1)
    idx = jnp.take_along_axis(lookup_indices, order, axis=-1).reshape(_COUT, 27)
    cf = lookup_coefficients.reshape(_COUT, 27)
    idx_p = jnp.zeros((256, 32), jnp.int32).at[:_COUT, :27].set(idx)
    cf_p = jnp.zeros((256, 32), jnp.float32).at[:_COUT, :27].set(cf)

    m_flat = _sc_build_m(idx_p, cf_p)
    mt = m_flat.reshape(256, 256)[:_COUT, :].T

    dict_r = jnp.transpose(dictionary, (2, 3, 1, 0)).reshape(9 * _CIN, _DICT)

    xt = jnp.transpose(x, (0, 2, 3, 1))
    xp = jnp.pad(xt, ((0, 0), (1, 1), (1, 1), (0, 0)))
    r0 = xp[:, 0::2, :, :]
    r1 = xp[:, 1::2, :, :]
    pr0 = r0.reshape(_B, 113, 113, 2 * _CIN)
    pr1 = r1.reshape(_B, 113, 113, 2 * _CIN)
    pr0a, pr0b, pr1a = pr0[:, :112], pr0[:, 1:], pr1[:, :112]
    q0 = r0[:, :, 2::2, :]
    q1 = r1[:, :, 2::2, :]
    q0a, q0b, q1a = q0[:, :112], q0[:, 1:], q1[:, :112]

    out_nhwc = _conv_tc(mt, dict_r, pr0a, pr1a, pr0b, q0a, q1a, q0b)
    return jnp.transpose(out_nhwc, (0, 3, 1, 2))


# speedup vs baseline: 1.6452x; 1.6452x over previous
import functools

import jax
import jax.numpy as jnp
from jax import lax
from jax.experimental import pallas as pl
from jax.experimental.pallas import tpu as pltpu
from jax.experimental.pallas import tpu_sc as plsc

_B, _CIN, _H, _W = 4, 96, 224, 224
_COUT, _KH, _KW, _SP = 192, 3, 3, 3
_DICT = 256
_HO, _WO = 112, 112
_BH = 16
_NH = _HO // _BH


def _sc_build_m(idx_p, cf_p):
    mesh = plsc.VectorSubcoreMesh(core_axis_name="c", subcore_axis_name="s")

    @functools.partial(
        pl.kernel,
        mesh=mesh,
        compiler_params=pltpu.CompilerParams(needs_layout_passes=False),
        out_type=jax.ShapeDtypeStruct((256 * 256,), jnp.float32),
        scratch_types=[
            pltpu.VMEM((8, 32), jnp.int32),
            pltpu.VMEM((8, 32), jnp.float32),
            pltpu.VMEM((8 * 256,), jnp.float32),
        ],
    )
    def build(idx_hbm, cf_hbm, m_hbm, idx_v, cf_v, buf):
        wid = lax.axis_index("s") * 2 + lax.axis_index("c")
        base = wid * 8
        pltpu.sync_copy(idx_hbm.at[pl.ds(base, 8)], idx_v)
        pltpu.sync_copy(cf_hbm.at[pl.ds(base, 8)], cf_v)
        z = jnp.zeros((16,), jnp.float32)
        for i in range(128):
            buf[pl.ds(i * 16, 16)] = z
        lanes = lax.broadcasted_iota(jnp.int32, (16,), 0)
        masks = [lanes == j for j in range(16)]
        for r in range(8):
            for v in range(2):
                iv = idx_v[r, pl.ds(v * 16, 16)] + r * 256
                cv = cf_v[r, pl.ds(v * 16, 16)]
                for j in range(16):
                    plsc.addupdate_scatter(buf, [iv], cv, mask=masks[j])
        pltpu.sync_copy(buf, m_hbm.at[pl.ds(wid * 2048, 2048)])

    return build(idx_p, cf_p)


def _conv_body(mt_ref, dr_ref, pa_ref, pb_ref, out_ref, w_ref):
    b = pl.program_id(0)
    h = pl.program_id(1)

    @pl.when((b == 0) & (h == 0))
    def _():
        w_ref[...] = jnp.dot(
            dr_ref[...], mt_ref[...], preferred_element_type=jnp.float32
        )

    a = pa_ref[0]
    ev = a[:, 0]
    od = a[:, 1]
    ev2 = jnp.concatenate([a[1:, 0], pb_ref[0, 0:1, 0]], axis=0)

    acc = jnp.zeros((_BH * _WO, _COUT), jnp.float32)
    for kh, g in ((0, ev), (1, od), (2, ev2)):
        xp_ = g[:, 0:_WO, :].reshape(_BH * _WO, 2 * _CIN)
        acc = acc + jnp.dot(
            xp_,
            w_ref[kh * 288 : kh * 288 + 192, :],
            preferred_element_type=jnp.float32,
        )
        xq = g[:, 1:113, 0:_CIN].reshape(_BH * _WO, _CIN)
        acc = acc + jnp.dot(
            xq,
            w_ref[kh * 288 + 192 : kh * 288 + 288, :],
            preferred_element_type=jnp.float32,
        )
    out_ref[0] = acc.reshape(_BH, _WO, _COUT)


def _conv_tc(mt, dict_r, pp):
    grid = (_B, _NH)
    const2 = lambda b, h: (0, 0)
    p_spec_a = pl.BlockSpec((1, _BH, 2, 113, 2 * _CIN), lambda b, h: (b, h, 0, 0, 0))
    p_spec_b = pl.BlockSpec(
        (1, _BH, 2, 113, 2 * _CIN),
        lambda b, h: (b, jnp.minimum(h + 1, 7), 0, 0, 0),
    )
    return pl.pallas_call(
        _conv_body,
        grid=grid,
        in_specs=[
            pl.BlockSpec((_DICT, _COUT), const2),
            pl.BlockSpec((9 * _CIN, _DICT), const2),
            p_spec_a,
            p_spec_b,
        ],
        out_specs=pl.BlockSpec((1, _BH, _WO, _COUT), lambda b, h: (b, h, 0, 0)),
        out_shape=jax.ShapeDtypeStruct((_B, _HO, _WO, _COUT), jnp.float32),
        scratch_shapes=[pltpu.VMEM((9 * _CIN, _COUT), jnp.float32)],
    )(mt, dict_r, pp, pp)


def kernel(x, dictionary, lookup_indices, lookup_coefficients):
    order = jnp.argsort(-jnp.abs(lookup_coefficients), axis=-1)
    idx = jnp.take_along_axis(lookup_indices, order, axis=-1).reshape(_COUT, 27)
    cf = lookup_coefficients.reshape(_COUT, 27)
    idx_p = jnp.zeros((256, 32), jnp.int32).at[:_COUT, :27].set(idx)
    cf_p = jnp.zeros((256, 32), jnp.float32).at[:_COUT, :27].set(cf)

    m_flat = _sc_build_m(idx_p, cf_p)
    mt = m_flat.reshape(256, 256)[:_COUT, :].T

    dict_r = jnp.transpose(dictionary, (2, 3, 1, 0)).reshape(9 * _CIN, _DICT)

    xt = jnp.transpose(x, (0, 2, 3, 1))
    xp = jnp.pad(xt, ((0, 0), (1, 1), (1, 1), (0, 0)))
    pp = xp.reshape(_B, 113, 2, 113, 2 * _CIN)

    out_nhwc = _conv_tc(mt, dict_r, pp)
    return jnp.transpose(out_nhwc, (0, 3, 1, 2))


# speedup vs baseline: 4.1144x; 2.5009x over previous
import functools

import jax
import jax.numpy as jnp
from jax import lax
from jax.experimental import pallas as pl
from jax.experimental.pallas import tpu as pltpu
from jax.experimental.pallas import tpu_sc as plsc

_B, _CIN, _H, _W = 4, 96, 224, 224
_COUT, _KH, _KW, _SP = 192, 3, 3, 3
_DICT = 256
_HO, _WO = 112, 112
_BH = 16
_NH = _HO // _BH


def _sc_build_m(idx_p, cf_p):
    mesh = plsc.VectorSubcoreMesh(core_axis_name="c", subcore_axis_name="s")

    @functools.partial(
        pl.kernel,
        mesh=mesh,
        compiler_params=pltpu.CompilerParams(needs_layout_passes=False),
        out_type=jax.ShapeDtypeStruct((256 * 256,), jnp.float32),
        scratch_types=[
            pltpu.VMEM((8, 32), jnp.int32),
            pltpu.VMEM((8, 32), jnp.float32),
            pltpu.VMEM((8 * 256,), jnp.float32),
        ],
    )
    def build(idx_hbm, cf_hbm, m_hbm, idx_v, cf_v, buf):
        wid = lax.axis_index("s") * 2 + lax.axis_index("c")
        base = wid * 8
        pltpu.sync_copy(idx_hbm.at[pl.ds(base, 8)], idx_v)
        pltpu.sync_copy(cf_hbm.at[pl.ds(base, 8)], cf_v)
        z = jnp.zeros((16,), jnp.float32)
        for i in range(128):
            buf[pl.ds(i * 16, 16)] = z
        lanes = lax.broadcasted_iota(jnp.int32, (16,), 0)
        masks = [lanes == j for j in range(16)]
        for r in range(8):
            for v in range(2):
                iv = idx_v[r, pl.ds(v * 16, 16)] + r * 256
                cv = cf_v[r, pl.ds(v * 16, 16)]
                for j in range(16):
                    plsc.addupdate_scatter(buf, [iv], cv, mask=masks[j])
        pltpu.sync_copy(buf, m_hbm.at[pl.ds(wid * 2048, 2048)])

    return build(idx_p, cf_p)


def _conv_body(mt_ref, dr_ref, pa_ref, pb_ref, out_ref, wb_ref):
    b = pl.program_id(0)
    h = pl.program_id(1)

    @pl.when((b == 0) & (h == 0))
    def _():
        w = jnp.dot(dr_ref[...], mt_ref[...], preferred_element_type=jnp.float32)
        wb_ref[...] = w.astype(jnp.bfloat16)

    a = pa_ref[0]
    ev = a[:, 0]
    od = a[:, 1]
    ev2 = jnp.concatenate([a[1:, 0], pb_ref[0, 0:1, 0]], axis=0)

    acc = jnp.zeros((_BH * _WO, _COUT), jnp.float32)
    for kh, g in ((0, ev), (1, od), (2, ev2)):
        xp_ = g[:, 0:_WO, :].reshape(_BH * _WO, 2 * _CIN)
        acc = acc + jnp.dot(
            xp_,
            wb_ref[kh * 288 : kh * 288 + 192, :],
            preferred_element_type=jnp.float32,
        )
        xq = g[:, 1:113, 0:_CIN].reshape(_BH * _WO, _CIN)
        acc = acc + jnp.dot(
            xq,
            wb_ref[kh * 288 + 192 : kh * 288 + 288, :],
            preferred_element_type=jnp.float32,
        )
    accr = acc.reshape(_BH, _WO, _COUT)
    for r in range(_BH):
        out_ref[0, :, r, :] = jnp.transpose(accr[r])


def _conv_tc(mt, dict_r, pp):
    grid = (_B, _NH)
    const2 = lambda b, h: (0, 0)
    p_spec_a = pl.BlockSpec((1, _BH, 2, 113, 2 * _CIN), lambda b, h: (b, h, 0, 0, 0))
    p_spec_b = pl.BlockSpec(
        (1, _BH, 2, 113, 2 * _CIN),
        lambda b, h: (b, jnp.minimum(h + 1, 7), 0, 0, 0),
    )
    return pl.pallas_call(
        _conv_body,
        grid=grid,
        in_specs=[
            pl.BlockSpec((_DICT, _COUT), const2),
            pl.BlockSpec((9 * _CIN, _DICT), const2),
            p_spec_a,
            p_spec_b,
        ],
        out_specs=pl.BlockSpec((1, _COUT, _BH, _WO), lambda b, h: (b, 0, h, 0)),
        out_shape=jax.ShapeDtypeStruct((_B, _COUT, _HO, _WO), jnp.float32),
        scratch_shapes=[pltpu.VMEM((9 * _CIN, _COUT), jnp.bfloat16)],
    )(mt, dict_r, pp, pp)


def kernel(x, dictionary, lookup_indices, lookup_coefficients):
    order = jnp.argsort(-jnp.abs(lookup_coefficients), axis=-1)
    idx = jnp.take_along_axis(lookup_indices, order, axis=-1).reshape(_COUT, 27)
    cf = lookup_coefficients.reshape(_COUT, 27)
    idx_p = jnp.zeros((256, 32), jnp.int32).at[:_COUT, :27].set(idx)
    cf_p = jnp.zeros((256, 32), jnp.float32).at[:_COUT, :27].set(cf)

    m_flat = _sc_build_m(idx_p, cf_p)
    mt = m_flat.reshape(256, 256)[:_COUT, :].T

    dict_r = jnp.transpose(dictionary, (2, 3, 1, 0)).reshape(9 * _CIN, _DICT)

    xt = jnp.transpose(x, (0, 2, 3, 1)).astype(jnp.bfloat16)
    xp = jnp.pad(xt, ((0, 0), (1, 1), (1, 1), (0, 0)))
    pp = xp.reshape(_B, 113, 2, 113, 2 * _CIN)

    return _conv_tc(mt, dict_r, pp)


# speedup vs baseline: 4.1710x; 1.0138x over previous
import functools

import jax
import jax.numpy as jnp
from jax import lax
from jax.experimental import pallas as pl
from jax.experimental.pallas import tpu as pltpu
from jax.experimental.pallas import tpu_sc as plsc

_B, _CIN, _H, _W = 4, 96, 224, 224
_COUT, _KH, _KW, _SP = 192, 3, 3, 3
_DICT = 256
_HO, _WO = 112, 112
_BH = 16
_NH = _HO // _BH


def _sc_build_m(idx_p, cf_p):
    mesh = plsc.VectorSubcoreMesh(core_axis_name="c", subcore_axis_name="s")

    @functools.partial(
        pl.kernel,
        mesh=mesh,
        compiler_params=pltpu.CompilerParams(needs_layout_passes=False),
        out_type=jax.ShapeDtypeStruct((256 * 256,), jnp.float32),
        scratch_types=[
            pltpu.VMEM((8, 32), jnp.int32),
            pltpu.VMEM((8, 32), jnp.float32),
            pltpu.VMEM((8 * 256,), jnp.float32),
        ],
    )
    def build(idx_hbm, cf_hbm, m_hbm, idx_v, cf_v, buf):
        wid = lax.axis_index("s") * 2 + lax.axis_index("c")
        base = wid * 8
        pltpu.sync_copy(idx_hbm.at[pl.ds(base, 8)], idx_v)
        pltpu.sync_copy(cf_hbm.at[pl.ds(base, 8)], cf_v)
        z = jnp.zeros((16,), jnp.float32)
        for i in range(128):
            buf[pl.ds(i * 16, 16)] = z
        lanes = lax.broadcasted_iota(jnp.int32, (16,), 0)
        masks = [lanes == j for j in range(16)]
        for r in range(8):
            for v in range(2):
                iv = idx_v[r, pl.ds(v * 16, 16)] + r * 256
                cv = cf_v[r, pl.ds(v * 16, 16)]
                for j in range(16):
                    plsc.addupdate_scatter(buf, [iv], cv, mask=masks[j])
        pltpu.sync_copy(buf, m_hbm.at[pl.ds(wid * 2048, 2048)])

    return build(idx_p, cf_p)


def _conv_body(mt_ref, dr_ref, pa_ref, pb_ref, out_ref, wb_ref):
    b = pl.program_id(0)
    h = pl.program_id(1)

    @pl.when((b == 0) & (h == 0))
    def _():
        w = jnp.dot(dr_ref[...], mt_ref[...], preferred_element_type=jnp.float32)
        wb_ref[...] = w.astype(jnp.bfloat16)

    a = pa_ref[0]
    ev = a[:, 0]
    od = a[:, 1]
    ev2 = jnp.concatenate([a[1:, 0], pb_ref[0, 0:1, 0]], axis=0)

    acc = jnp.zeros((_BH * _WO, _COUT), jnp.float32)
    for kh, g in ((0, ev), (1, od), (2, ev2)):
        xp_ = g[:, 0:_WO, :].reshape(_BH * _WO, 2 * _CIN)
        acc = acc + jnp.dot(
            xp_,
            wb_ref[kh * 288 : kh * 288 + 192, :],
            preferred_element_type=jnp.float32,
        )
        xq = g[:, 1:113, 0:_CIN].reshape(_BH * _WO, _CIN)
        acc = acc + jnp.dot(
            xq,
            wb_ref[kh * 288 + 192 : kh * 288 + 288, :],
            preferred_element_type=jnp.float32,
        )
    accr = acc.reshape(_BH, _WO, _COUT)
    for r in range(_BH):
        out_ref[0, :, r, :] = jnp.transpose(accr[r])


def _conv_tc(mt, dict_r, pp):
    grid = (_B, _NH)
    const2 = lambda b, h: (0, 0)
    p_spec_a = pl.BlockSpec((1, _BH, 2, 113, 2 * _CIN), lambda b, h: (b, h, 0, 0, 0))
    p_spec_b = pl.BlockSpec(
        (1, 4, 2, 113, 2 * _CIN),
        lambda b, h: (b, jnp.minimum(4 * h + 4, 28), 0, 0, 0),
    )
    return pl.pallas_call(
        _conv_body,
        grid=grid,
        in_specs=[
            pl.BlockSpec((_DICT, _COUT), const2),
            pl.BlockSpec((9 * _CIN, _DICT), const2),
            p_spec_a,
            p_spec_b,
        ],
        out_specs=pl.BlockSpec((1, _COUT, _BH, _WO), lambda b, h: (b, 0, h, 0)),
        out_shape=jax.ShapeDtypeStruct((_B, _COUT, _HO, _WO), jnp.float32),
        scratch_shapes=[pltpu.VMEM((9 * _CIN, _COUT), jnp.bfloat16)],
    )(mt, dict_r, pp, pp)


def kernel(x, dictionary, lookup_indices, lookup_coefficients):
    order = jnp.argsort(-jnp.abs(lookup_coefficients), axis=-1)
    idx = jnp.take_along_axis(lookup_indices, order, axis=-1).reshape(_COUT, 27)
    cf = lookup_coefficients.reshape(_COUT, 27)
    idx_p = jnp.zeros((256, 32), jnp.int32).at[:_COUT, :27].set(idx)
    cf_p = jnp.zeros((256, 32), jnp.float32).at[:_COUT, :27].set(cf)

    m_flat = _sc_build_m(idx_p, cf_p)
    mt = m_flat.reshape(256, 256)[:_COUT, :].T

    dict_r = jnp.transpose(dictionary, (2, 3, 1, 0)).reshape(9 * _CIN, _DICT)

    xt = jnp.transpose(x, (0, 2, 3, 1)).astype(jnp.bfloat16)
    xp = jnp.pad(xt, ((0, 0), (1, 1), (1, 1), (0, 0)))
    pp = xp.reshape(_B, 113, 2, 113, 2 * _CIN)

    return _conv_tc(mt, dict_r, pp)
